# Initial kernel scaffold; baseline (speedup 1.0000x reference)
import jax
import jax.numpy as jnp
from jax.experimental import pallas as pl
from jax.experimental.pallas import tpu as pltpu

N = 2048
G = 32
C0 = 64
F = 32


def _bn_relu_cols(out, gamma, beta):
    n_tot = jnp.float32(G * N)
    s = jnp.sum(out, axis=0)
    ss = jnp.sum(out * out, axis=0)
    s_f = jnp.sum(s.reshape(G, F), axis=0)
    ss_f = jnp.sum(ss.reshape(G, F), axis=0)
    mu = s_f / n_tot
    var = ss_f / n_tot - mu * mu
    scale = gamma * jax.lax.rsqrt(var + 1e-5)
    shift = beta - mu * scale
    colscale = jnp.broadcast_to(scale[None, :], (G, F)).reshape(1, G * F)
    colshift = jnp.broadcast_to(shift[None, :], (G, F)).reshape(1, G * F)
    return jnp.maximum(out * colscale + colshift, 0.0)


def _gcn3_body(h0_ref, a_ref, d2_ref, w1_ref, w2_ref, w3_ref, b_ref, g_ref,
               be_ref, out_ref):
    A = a_ref[...]
    d2 = d2_ref[...].T
    H = h0_ref[...]

    def colvec(ref, i):
        return jnp.broadcast_to(ref[i, :][None, :], (G, F)).reshape(1, G * F)

    hw = jnp.dot(H.reshape(N * G, C0), w1_ref[...],
                 preferred_element_type=jnp.float32).reshape(N, G * F)
    out = jnp.dot(A, hw, preferred_element_type=jnp.float32)
    out = out + d2 * hw + colvec(b_ref, 0)
    H = _bn_relu_cols(out, g_ref[0, :], be_ref[0, :])
    for i, w_ref in ((1, w2_ref), (2, w3_ref)):
        hw = jnp.dot(H.reshape(N * G, F), w_ref[...],
                     preferred_element_type=jnp.float32).reshape(N, G * F)
        out = jnp.dot(A, hw, preferred_element_type=jnp.float32)
        out = out + d2 * hw + colvec(b_ref, i)
        H = _bn_relu_cols(out, g_ref[i, :], be_ref[i, :])
    out_ref[...] = H


def _gcn3(h0, a, d2, w1, w2, w3, b, g, be):
    return pl.pallas_call(
        _gcn3_body,
        out_shape=jax.ShapeDtypeStruct((N, G * F), jnp.float32),
    )(h0, a, d2, w1, w2, w3, b, g, be)


def _build_a(edge_index):
    src, dst = edge_index[0], edge_index[1]
    deg = jnp.zeros((N,), jnp.float32).at[dst].add(1.0) + 2.0
    dis = jax.lax.rsqrt(deg)
    norm = dis[src] * dis[dst]
    a = jnp.zeros((N, N), jnp.float32).at[dst, src].add(norm)
    return a, deg


def kernel(x, edge_index, W1, b1, g1, be1, W2, b2, g2, be2, W3, b3, g3, be3):
    a, deg = _build_a(edge_index)
    d2 = (2.0 / deg).reshape(1, N)
    h0 = x.reshape(G * C0, N).T
    b = jnp.stack([b1, b2, b3])
    g = jnp.stack([g1, g2, g3])
    be = jnp.stack([be1, be2, be3])
    h3 = _gcn3(h0, a, d2, W1, W2, W3, b, g, be)
    return h3.reshape(N, G, F).transpose(1, 2, 0)


# speedup vs baseline: 24.6098x; 24.6098x over previous
import jax
import jax.numpy as jnp
from jax.experimental import pallas as pl
from jax.experimental.pallas import tpu as pltpu

N = 2048
G = 32
C0 = 64
F = 32


def _mm_body(x_ref, w_ref, o_ref):
    o_ref[...] = jnp.dot(x_ref[...], w_ref[...],
                         preferred_element_type=jnp.float32)


def _mm(x, w):
    return pl.pallas_call(
        _mm_body,
        out_shape=jax.ShapeDtypeStruct((x.shape[0], w.shape[1]), jnp.float32),
    )(x, w)


def _agg_body(a_ref, hw_ref, d2_ref, gam_ref, bet_ref, t_ref, tt_ref, o_ref):
    hw = hw_ref[...]
    out = jnp.dot(a_ref[...], hw, preferred_element_type=jnp.float32)
    out = out + d2_ref[...] * hw
    n_tot = jnp.float32(G * N)
    t = t_ref[...]
    s = jnp.sum(out, axis=0, keepdims=True)
    ss = jnp.sum(out * out, axis=0, keepdims=True)
    s_f = jnp.dot(s, t, preferred_element_type=jnp.float32)
    ss_f = jnp.dot(ss, t, preferred_element_type=jnp.float32)
    mu = s_f / n_tot
    var = ss_f / n_tot - mu * mu
    scale = gam_ref[...] * jax.lax.rsqrt(var + 1e-5)
    shift = bet_ref[...] - mu * scale
    colscale = jnp.dot(scale, tt_ref[...], preferred_element_type=jnp.float32)
    colshift = jnp.dot(shift, tt_ref[...], preferred_element_type=jnp.float32)
    o_ref[...] = jnp.maximum(out * colscale + colshift, 0.0)


def _agg(a, hw, d2, gam, bet, t, tt):
    return pl.pallas_call(
        _agg_body,
        out_shape=jax.ShapeDtypeStruct((N, G * F), jnp.float32),
    )(a, hw, d2, gam, bet, t, tt)


def _build_a(edge_index):
    src, dst = edge_index[0], edge_index[1]
    deg = jnp.zeros((N,), jnp.float32).at[dst].add(1.0) + 2.0
    dis = jax.lax.rsqrt(deg)
    norm = dis[src] * dis[dst]
    a = jnp.zeros((N, N), jnp.float32).at[dst, src].add(norm)
    return a, deg


def _kron_eye(w):
    ci, co = w.shape
    eye = jnp.eye(G, dtype=w.dtype)
    return (eye[:, None, :, None] * w[None, :, None, :]).reshape(G * ci, G * co)


def kernel(x, edge_index, W1, b1, g1, be1, W2, b2, g2, be2, W3, b3, g3, be3):
    a, deg = _build_a(edge_index)
    d2 = (2.0 / deg).reshape(N, 1)
    h0 = x.reshape(G * C0, N).T
    t = jnp.tile(jnp.eye(F, dtype=jnp.float32), (G, 1))
    tt = t.T
    H = h0
    for w, gam, bet in ((W1, g1, be1), (W2, g2, be2), (W3, g3, be3)):
        hw = _mm(H, _kron_eye(w))
        H = _agg(a, hw, d2, gam.reshape(1, F), bet.reshape(1, F), t, tt)
    return H.reshape(N, G, F).transpose(1, 2, 0)


# speedup vs baseline: 97.6235x; 3.9669x over previous
import dataclasses
import functools

import jax
import jax.numpy as jnp
from jax import lax
from jax.experimental import pallas as pl
from jax.experimental.pallas import tpu as pltpu
from jax.experimental.pallas import tpu_sc as plsc

N = 2048
G = 32
C0 = 64
F = 32
E = 65536

NS = 16
NW = 2 * NS
ROWS_W = N // NW
HALF_ROWS = ROWS_W // 2
HALF_W = HALF_ROWS * N
EPIECE = 4096
ZW = 8192


def _build_c(src, dst):
    cp = pltpu.CompilerParams()
    if "needs_layout_passes" in pltpu.CompilerParams.__dataclass_fields__:
        cp = dataclasses.replace(cp, needs_layout_passes=False)

    @functools.partial(
        pl.kernel,
        out_type=jax.ShapeDtypeStruct((N * N,), jnp.float32),
        mesh=plsc.VectorSubcoreMesh(core_axis_name="c", subcore_axis_name="s"),
        compiler_params=cp,
        scratch_types=[
            pltpu.VMEM((HALF_W,), jnp.float32),
            pltpu.VMEM((EPIECE,), jnp.int32),
            pltpu.VMEM((EPIECE,), jnp.int32),
            pltpu.SemaphoreType.DMA,
        ],
    )
    def k(src_hbm, dst_hbm, c_hbm, cbuf, srcb, dstb, sem):
        wid = lax.axis_index("c") * NS + lax.axis_index("s")
        zero16 = jnp.zeros((16,), jnp.float32)
        one16 = jnp.ones((16,), jnp.float32)

        @pl.loop(0, 2)
        def _(half):
            base_row = wid * ROWS_W + half * HALF_ROWS

            @pl.loop(0, HALF_W, step=16)
            def _(off):
                cbuf[pl.ds(off, 16)] = zero16

            @pl.loop(0, E, step=EPIECE)
            def _(e0):
                pltpu.async_copy(src_hbm.at[pl.ds(e0, EPIECE)], srcb, sem).wait()
                pltpu.async_copy(dst_hbm.at[pl.ds(e0, EPIECE)], dstb, sem).wait()

                @pl.loop(0, EPIECE, step=16)
                def _(v):
                    dv = dstb[pl.ds(v, 16)]
                    sv = srcb[pl.ds(v, 16)]
                    r = dv - base_row
                    mask = (r >= 0) & (r < HALF_ROWS)
                    idx = r * N + sv
                    plsc.addupdate_scatter(cbuf, [idx], one16, mask=mask)

            pltpu.async_copy(
                cbuf, c_hbm.at[pl.ds(base_row * N, HALF_W)], sem
            ).wait()

    return k(src, dst)


def _mm_body(x_ref, w_ref, o_ref):
    o_ref[...] = jnp.dot(x_ref[...], w_ref[...],
                         preferred_element_type=jnp.float32)


def _mm(x, w):
    return pl.pallas_call(
        _mm_body,
        out_shape=jax.ShapeDtypeStruct((x.shape[0], w.shape[1]), jnp.float32),
    )(x, w)


def _agg_body(c_ref, hw_ref, gam_ref, bet_ref, t_ref, tt_ref, o_ref):
    c = c_ref[...]
    hw = hw_ref[...]
    deg = jnp.sum(c, axis=1, keepdims=True) + 2.0
    dis = jax.lax.rsqrt(deg)
    out = jnp.dot(c, dis * hw, preferred_element_type=jnp.float32)
    out = dis * out + (2.0 / deg) * hw
    n_tot = jnp.float32(G * N)
    t = t_ref[...]
    s = jnp.sum(out, axis=0, keepdims=True)
    ss = jnp.sum(out * out, axis=0, keepdims=True)
    s_f = jnp.dot(s, t, preferred_element_type=jnp.float32)
    ss_f = jnp.dot(ss, t, preferred_element_type=jnp.float32)
    mu = s_f / n_tot
    var = ss_f / n_tot - mu * mu
    scale = gam_ref[...] * jax.lax.rsqrt(var + 1e-5)
    shift = bet_ref[...] - mu * scale
    colscale = jnp.dot(scale, tt_ref[...], preferred_element_type=jnp.float32)
    colshift = jnp.dot(shift, tt_ref[...], preferred_element_type=jnp.float32)
    o_ref[...] = jnp.maximum(out * colscale + colshift, 0.0)


def _agg(c, hw, gam, bet, t, tt):
    return pl.pallas_call(
        _agg_body,
        out_shape=jax.ShapeDtypeStruct((N, G * F), jnp.float32),
    )(c, hw, gam, bet, t, tt)


def _kron_eye(w):
    ci, co = w.shape
    eye = jnp.eye(G, dtype=w.dtype)
    return (eye[:, None, :, None] * w[None, :, None, :]).reshape(G * ci, G * co)


def kernel(x, edge_index, W1, b1, g1, be1, W2, b2, g2, be2, W3, b3, g3, be3):
    c = _build_c(edge_index[0], edge_index[1]).reshape(N, N)
    h0 = x.reshape(G * C0, N).T
    t = jnp.tile(jnp.eye(F, dtype=jnp.float32), (G, 1))
    tt = t.T
    H = h0
    for w, gam, bet in ((W1, g1, be1), (W2, g2, be2), (W3, g3, be3)):
        hw = _mm(H, _kron_eye(w))
        H = _agg(c, hw, gam.reshape(1, F), bet.reshape(1, F), t, tt)
    return H.reshape(N, G, F).transpose(1, 2, 0)


# speedup vs baseline: 134.2283x; 1.3750x over previous
import dataclasses
import functools

import jax
import jax.numpy as jnp
from jax import lax
from jax.experimental import pallas as pl
from jax.experimental.pallas import tpu as pltpu
from jax.experimental.pallas import tpu_sc as plsc

N = 2048
G = 32
C0 = 64
F = 32
E = 65536

NS = 16
NW = 2 * NS
ROWS_W = N // NW
HALF_ROWS = ROWS_W // 2
HALF_W = HALF_ROWS * N
EPIECE = 16384
NPIECE = E // EPIECE


def _build_c(eidx):
    cp = pltpu.CompilerParams()
    if "needs_layout_passes" in pltpu.CompilerParams.__dataclass_fields__:
        cp = dataclasses.replace(cp, needs_layout_passes=False)

    @functools.partial(
        pl.kernel,
        out_type=jax.ShapeDtypeStruct((N * N,), jnp.float32),
        mesh=plsc.VectorSubcoreMesh(core_axis_name="c", subcore_axis_name="s"),
        compiler_params=cp,
        scratch_types=[
            pltpu.VMEM((HALF_W,), jnp.float32),
            pltpu.VMEM((EPIECE,), jnp.int32),
            pltpu.VMEM((EPIECE,), jnp.int32),
            pltpu.SemaphoreType.DMA,
            pltpu.SemaphoreType.DMA,
            pltpu.SemaphoreType.DMA,
        ],
    )
    def k(eidx_hbm, c_hbm, cbuf, eb0, eb1, sem0, sem1, semo):
        wid = lax.axis_index("c") * NS + lax.axis_index("s")
        zero16 = jnp.zeros((16,), jnp.float32)
        one16 = jnp.ones((16,), jnp.float32)
        bufs = (eb0, eb1)
        sems = (sem0, sem1)

        @pl.loop(0, 2)
        def _(half):
            basew = (wid * ROWS_W + half * HALF_ROWS) * N

            @pl.loop(0, HALF_W, step=64)
            def _(off):
                for j in range(0, 64, 16):
                    cbuf[pl.ds(off + j, 16)] = zero16

            cps = [None] * NPIECE
            cps[0] = pltpu.async_copy(
                eidx_hbm.at[pl.ds(0, EPIECE)], eb0, sem0)
            for i in range(NPIECE):
                if i + 1 < NPIECE:
                    cps[i + 1] = pltpu.async_copy(
                        eidx_hbm.at[pl.ds((i + 1) * EPIECE, EPIECE)],
                        bufs[(i + 1) % 2], sems[(i + 1) % 2])
                cps[i].wait()
                buf = bufs[i % 2]

                @pl.loop(0, EPIECE, step=64)
                def _(v, buf=buf):
                    for j in range(0, 64, 16):
                        r = buf[pl.ds(v + j, 16)] - basew
                        mask = (r >= 0) & (r < HALF_W)
                        plsc.addupdate_scatter(cbuf, [r], one16, mask=mask)

            pltpu.async_copy(
                cbuf, c_hbm.at[pl.ds(basew, HALF_W)], semo
            ).wait()

    return k(eidx)


def _mm_body(x_ref, w_ref, o_ref):
    o_ref[...] = jnp.dot(x_ref[...], w_ref[...],
                         preferred_element_type=jnp.float32)


def _mm(x, w):
    return pl.pallas_call(
        _mm_body,
        out_shape=jax.ShapeDtypeStruct((x.shape[0], w.shape[1]), jnp.float32),
    )(x, w)


def _agg_body(c_ref, hw_ref, gam_ref, bet_ref, t_ref, tt_ref, o_ref):
    c = c_ref[...]
    hw = hw_ref[...]
    deg = jnp.sum(c, axis=1, keepdims=True) + 2.0
    dis = jax.lax.rsqrt(deg)
    out = jnp.dot(c, dis * hw, preferred_element_type=jnp.float32)
    out = dis * out + (2.0 / deg) * hw
    n_tot = jnp.float32(G * N)
    t = t_ref[...]
    s = jnp.sum(out, axis=0, keepdims=True)
    ss = jnp.sum(out * out, axis=0, keepdims=True)
    s_f = jnp.dot(s, t, preferred_element_type=jnp.float32)
    ss_f = jnp.dot(ss, t, preferred_element_type=jnp.float32)
    mu = s_f / n_tot
    var = ss_f / n_tot - mu * mu
    scale = gam_ref[...] * jax.lax.rsqrt(var + 1e-5)
    shift = bet_ref[...] - mu * scale
    colscale = jnp.dot(scale, tt_ref[...], preferred_element_type=jnp.float32)
    colshift = jnp.dot(shift, tt_ref[...], preferred_element_type=jnp.float32)
    o_ref[...] = jnp.maximum(out * colscale + colshift, 0.0)


def _agg(c, hw, gam, bet, t, tt):
    return pl.pallas_call(
        _agg_body,
        out_shape=jax.ShapeDtypeStruct((N, G * F), jnp.float32),
    )(c, hw, gam, bet, t, tt)


def _kron_eye(w):
    ci, co = w.shape
    eye = jnp.eye(G, dtype=w.dtype)
    return (eye[:, None, :, None] * w[None, :, None, :]).reshape(G * ci, G * co)


def kernel(x, edge_index, W1, b1, g1, be1, W2, b2, g2, be2, W3, b3, g3, be3):
    eidx = edge_index[1] * N + edge_index[0]
    c = _build_c(eidx).reshape(N, N)
    h0 = x.reshape(G * C0, N).T
    t = jnp.tile(jnp.eye(F, dtype=jnp.float32), (G, 1))
    tt = t.T
    H = h0
    for w, gam, bet in ((W1, g1, be1), (W2, g2, be2), (W3, g3, be3)):
        hw = _mm(H, _kron_eye(w))
        H = _agg(c, hw, gam.reshape(1, F), bet.reshape(1, F), t, tt)
    return H.reshape(N, G, F).transpose(1, 2, 0)
